# baseline (speedup 1.0000x reference)
import functools

import jax
import jax.numpy as jnp
from jax import lax
from jax.experimental import pallas as pl
from jax.experimental.pallas import tpu as pltpu
from jax.experimental.pallas import tpu_sc as plsc

B, S, D = 2, 2048, 768
H = 2 * D
A = H // 16
E = 8
G = 4
NG = max(1, E // G)
K = 2
N = B * S
TF = 512

SC_NC = 2
SC_NS = 16
SC_L = 16
SC_NW = SC_NC * SC_NS
SC_TPW = N // SC_NW


def _silu(x):
    return x * (0.5 * jnp.tanh(0.5 * x) + 0.5)


def _ln(x, g, b):
    mu = jnp.mean(x, axis=-1, keepdims=True)
    d = x - mu
    v = jnp.mean(d * d, axis=-1, keepdims=True)
    return d * jax.lax.rsqrt(v + 1e-5) * g + b


def _pre_body(wadapt_ref, wdown_ref, wep_ref, wop_ref, wawd_ref, wepwop_ref):
    wawd_ref[...] = jnp.dot(wadapt_ref[...], wdown_ref[...],
                            preferred_element_type=jnp.float32)
    wepwop_ref[...] = jnp.dot(wep_ref[...], wop_ref[...],
                              preferred_element_type=jnp.float32)


def _rlog_body(x_ref, wr_ref, rlogt_ref):
    rlogt_ref[...] = jax.lax.dot_general(
        wr_ref[...], x_ref[...], (((0,), (1,)), ((), ())),
        preferred_element_type=jnp.float32)


def _main_body(x_ref, wup_ref, wgate_ref, wdown_ref, wpre_ref, wpost_ref,
               gn_ref, bn_ref,
               sh_ref, pre_ref, ai_ref, ao_ref):
    xt = x_ref[...]
    xb = xt.astype(jnp.bfloat16)
    up = jnp.dot(xb, wup_ref[...], preferred_element_type=jnp.float32)
    gate = jnp.dot(xb, wgate_ref[...], preferred_element_type=jnp.float32)
    hidden = (_silu(gate) * up).astype(jnp.bfloat16)
    sh_ref[...] = jnp.dot(hidden, wdown_ref[...],
                          preferred_element_type=jnp.float32)
    pre = jnp.dot(xb, wpre_ref[...], preferred_element_type=jnp.float32)
    pre_ref[...] = pre
    g = gn_ref[...]
    b = bn_ref[...]
    ai_ref[...] = _ln(pre, g, b)
    po = jnp.dot(hidden, wpost_ref[...], preferred_element_type=jnp.float32)
    ao_ref[...] = _ln(po, g, b)


def _sc_router_body(rlogt_hbm, dmt_hbm, rin, rout):
    wid = lax.axis_index("s") * SC_NC + lax.axis_index("c")
    base = wid * SC_TPW
    pltpu.sync_copy(rlogt_hbm.at[:, pl.ds(base, SC_TPW)], rin)

    kvec = [jnp.full((SC_L,), k, jnp.int32) for k in range(G)]
    for grp in range(SC_TPW // SC_L):
        cols = pl.ds(grp * SC_L, SC_L)
        gl0 = rin[0, cols]
        gl1 = rin[1, cols]
        ll = [rin[2 + k, cols] for k in range(G)]
        gm = jnp.maximum(gl0, gl1)
        ex0 = jnp.exp(gl0 - gm)
        ex1 = jnp.exp(gl1 - gm)
        cw = jnp.maximum(ex0, ex1) / (ex0 + ex1)
        is1 = gl1 > gl0
        lm = jnp.maximum(jnp.maximum(ll[0], ll[1]), jnp.maximum(ll[2], ll[3]))
        le = [jnp.exp(l - lm) for l in ll]
        lsum = (le[0] + le[1]) + (le[2] + le[3])
        lp = [l / lsum for l in le]
        p0 = jnp.maximum(jnp.maximum(lp[0], lp[1]), jnp.maximum(lp[2], lp[3]))
        ti0 = jnp.where(lp[0] == p0, kvec[0],
                        jnp.where(lp[1] == p0, kvec[1],
                                  jnp.where(lp[2] == p0, kvec[2], kvec[3])))
        lpx = [jnp.where(ti0 == kvec[k], -1.0, lp[k]) for k in range(G)]
        p1 = jnp.maximum(jnp.maximum(lpx[0], lpx[1]),
                         jnp.maximum(lpx[2], lpx[3]))
        ti1 = jnp.where(lpx[0] == p1, kvec[0],
                        jnp.where(lpx[1] == p1, kvec[1],
                                  jnp.where(lpx[2] == p1, kvec[2], kvec[3])))
        inv = 1.0 / (p0 + p1 + 1e-7)
        fw0 = cw * p0 * inv
        fw1 = cw * p1 * inv
        for e in range(E):
            l_id = e % G
            wl = (jnp.where(ti0 == kvec[l_id], fw0, 0.0)
                  + jnp.where(ti1 == kvec[l_id], fw1, 0.0))
            if e // G == 1:
                dme = jnp.where(is1, wl, 0.0)
            else:
                dme = jnp.where(is1, 0.0, wl)
            rout[e, cols] = dme
    pltpu.sync_copy(rout, dmt_hbm.at[:, pl.ds(base, SC_TPW)])


def _tail_body(sh_ref, pre_ref, aib_ref, aob_ref, dmt_ref,
               adw_ref, adgf_ref, adb_ref, ub_ref, bb_ref,
               wawd_ref, wepwop_ref, out_ref):
    i = pl.program_id(1)
    rows = pl.ds(i * TF, TF)
    ai_b = aib_ref[0]
    ao_b = aob_ref[0]
    ai_t = aib_ref[0, rows, :]

    aw = jax.lax.dot_general(ai_t.astype(jnp.bfloat16),
                             ao_b.astype(jnp.bfloat16),
                             (((1,), (1,)), ((), ())),
                             preferred_element_type=jnp.float32)
    aw = _silu(jnp.clip(aw, -5.0, 5.0)).astype(jnp.bfloat16)
    ac = jnp.dot(aw, ai_b.astype(jnp.bfloat16),
                 preferred_element_type=jnp.float32)
    adapt_contrib = jnp.dot(ac, wawd_ref[...],
                            preferred_element_type=jnp.float32)
    shared = sh_ref[0] + 0.1 * adapt_contrib

    dm = jnp.transpose(dmt_ref[...])
    wsum = jnp.sum(dm, axis=1, keepdims=True)

    pre_t = pre_ref[0]
    P = jnp.dot(pre_t, adw_ref[...],
                preferred_element_type=jnp.float32)
    sums = jnp.dot(P, ub_ref[...], preferred_element_type=jnp.float32)
    sums2 = jnp.dot(P * P, ub_ref[...], preferred_element_type=jnp.float32)
    mu = sums * (1.0 / A)
    var = sums2 * (1.0 / A) - mu * mu
    rs = jax.lax.rsqrt(var + 1e-5)
    sc = dm * rs
    uc = sc * mu
    Sb = jnp.dot(sc, bb_ref[...], preferred_element_type=jnp.float32)
    Ub = jnp.dot(uc, bb_ref[...], preferred_element_type=jnp.float32)
    zfull = (P * Sb - Ub) * adgf_ref[...]
    z = zfull[:, 0:A]
    for e in range(1, E):
        z = z + zfull[:, e * A:(e + 1) * A]
    z = z + jnp.dot(dm, adb_ref[...], preferred_element_type=jnp.float32)
    expert = jnp.dot(z, wepwop_ref[...],
                     preferred_element_type=jnp.float32)
    out_ref[0] = shared * wsum + 0.1 * expert


@functools.partial(jax.jit, static_argnames=("interpret",))
def _run(x, Wup, Wgate, Wdown, Wpre, Wpost, g_norm, b_norm, Wadapt, adW,
         adg, adb, Wep, Wop, Wrg, Wre, interpret=False):
    xf = x.reshape(N, D)
    wr = jnp.concatenate(
        [Wrg, Wre, jnp.zeros((D, 8 - NG - G), jnp.float32)], axis=1)
    adw_all = jnp.transpose(adW, (1, 0, 2)).reshape(A, E * A)
    gn = g_norm.reshape(1, A)
    bn = b_norm.reshape(1, A)
    ub = jnp.repeat(jnp.eye(E, dtype=jnp.float32), A, axis=0)
    bb = ub.T
    adgf = adg.reshape(1, E * A)
    wup_b = Wup.astype(jnp.bfloat16)
    wgate_b = Wgate.astype(jnp.bfloat16)
    wdown_b = Wdown.astype(jnp.bfloat16)
    wpost_b = Wpost.astype(jnp.bfloat16)
    wpre_b = Wpre.astype(jnp.bfloat16)

    wawd, wepwop = pl.pallas_call(
        _pre_body,
        out_shape=(jax.ShapeDtypeStruct((A, D), jnp.float32),
                   jax.ShapeDtypeStruct((A, D), jnp.float32)),
        interpret=interpret,
    )(Wadapt, Wdown, Wep, Wop)

    nt = N // TF
    rlogt = pl.pallas_call(
        _rlog_body,
        grid=(nt,),
        in_specs=[
            pl.BlockSpec((TF, D), lambda i: (i, 0)),
            pl.BlockSpec((D, 8), lambda i: (0, 0)),
        ],
        out_specs=pl.BlockSpec((8, TF), lambda i: (0, i)),
        out_shape=jax.ShapeDtypeStruct((8, N), jnp.float32),
        interpret=interpret,
    )(xf, wr)

    sc_router = functools.partial(
        pl.kernel,
        mesh=plsc.VectorSubcoreMesh(core_axis_name="c", subcore_axis_name="s"),
        out_type=jax.ShapeDtypeStruct((E, N), jnp.float32),
        scratch_types=[
            pltpu.VMEM((8, SC_TPW), jnp.float32),
            pltpu.VMEM((E, SC_TPW), jnp.float32),
        ],
    )(_sc_router_body)
    dmt = sc_router(rlogt)

    sh, pre, ai, ao = pl.pallas_call(
        _main_body,
        grid=(nt,),
        in_specs=[
            pl.BlockSpec((TF, D), lambda i: (i, 0)),
            pl.BlockSpec((D, H), lambda i: (0, 0)),
            pl.BlockSpec((D, H), lambda i: (0, 0)),
            pl.BlockSpec((H, D), lambda i: (0, 0)),
            pl.BlockSpec((D, A), lambda i: (0, 0)),
            pl.BlockSpec((H, A), lambda i: (0, 0)),
            pl.BlockSpec((1, A), lambda i: (0, 0)),
            pl.BlockSpec((1, A), lambda i: (0, 0)),
        ],
        out_specs=[
            pl.BlockSpec((TF, D), lambda i: (i, 0)),
            pl.BlockSpec((TF, A), lambda i: (i, 0)),
            pl.BlockSpec((TF, A), lambda i: (i, 0)),
            pl.BlockSpec((TF, A), lambda i: (i, 0)),
        ],
        out_shape=(
            jax.ShapeDtypeStruct((N, D), jnp.float32),
            jax.ShapeDtypeStruct((N, A), jnp.float32),
            jax.ShapeDtypeStruct((N, A), jnp.float32),
            jax.ShapeDtypeStruct((N, A), jnp.float32),
        ),
        interpret=interpret,
    )(xf, wup_b, wgate_b, wdown_b, wpre_b, wpost_b, gn, bn)

    st = S // TF
    out = pl.pallas_call(
        _tail_body,
        grid=(B, st),
        in_specs=[
            pl.BlockSpec((1, TF, D), lambda b, i: (b, i, 0)),
            pl.BlockSpec((1, TF, A), lambda b, i: (b, i, 0)),
            pl.BlockSpec((1, S, A), lambda b, i: (b, 0, 0)),
            pl.BlockSpec((1, S, A), lambda b, i: (b, 0, 0)),
            pl.BlockSpec((E, TF), lambda b, i: (0, b * (S // TF) + i)),
            pl.BlockSpec((A, E * A), lambda b, i: (0, 0)),
            pl.BlockSpec((1, E * A), lambda b, i: (0, 0)),
            pl.BlockSpec((E, A), lambda b, i: (0, 0)),
            pl.BlockSpec((E * A, E), lambda b, i: (0, 0)),
            pl.BlockSpec((E, E * A), lambda b, i: (0, 0)),
            pl.BlockSpec((A, D), lambda b, i: (0, 0)),
            pl.BlockSpec((A, D), lambda b, i: (0, 0)),
        ],
        out_specs=pl.BlockSpec((1, TF, D), lambda b, i: (b, i, 0)),
        out_shape=jax.ShapeDtypeStruct((B, S, D), jnp.float32),
        interpret=interpret,
    )(sh.reshape(B, S, D), pre.reshape(B, S, A), ai.reshape(B, S, A),
      ao.reshape(B, S, A), dmt,
      adw_all, adgf, adb, ub, bb, wawd, wepwop)
    return out


def kernel(x, Wup, Wgate, Wdown, Wpre, Wpost, g_norm, b_norm, Wadapt, adW,
           adg, adb, Wep, Wop, Wrg, Wre):
    return _run(x, Wup, Wgate, Wdown, Wpre, Wpost, g_norm, b_norm, Wadapt,
                adW, adg, adb, Wep, Wop, Wrg, Wre)


# speedup vs baseline: 1.0241x; 1.0241x over previous
import functools

import jax
import jax.numpy as jnp
from jax import lax
from jax.experimental import pallas as pl
from jax.experimental.pallas import tpu as pltpu
from jax.experimental.pallas import tpu_sc as plsc

B, S, D = 2, 2048, 768
H = 2 * D
A = H // 16
E = 8
G = 4
NG = max(1, E // G)
K = 2
N = B * S
TF = 512

SC_NC = 2
SC_NS = 16
SC_L = 16
SC_NW = SC_NC * SC_NS
SC_TPW = N // SC_NW


def _silu(x):
    return x * (0.5 * jnp.tanh(0.5 * x) + 0.5)


def _ln(x, g, b):
    mu = jnp.mean(x, axis=-1, keepdims=True)
    d = x - mu
    v = jnp.mean(d * d, axis=-1, keepdims=True)
    return d * jax.lax.rsqrt(v + 1e-5) * g + b


def _pre_body(wadapt_ref, wdown_ref, wep_ref, wop_ref, wawd_ref, wepwop_ref):
    wawd_ref[...] = jnp.dot(wadapt_ref[...], wdown_ref[...],
                            preferred_element_type=jnp.float32)
    wepwop_ref[...] = jnp.dot(wep_ref[...], wop_ref[...],
                              preferred_element_type=jnp.float32)


def _rlog_body(x_ref, wr_ref, rlogt_ref):
    rlogt_ref[...] = jax.lax.dot_general(
        wr_ref[...], x_ref[...], (((0,), (1,)), ((), ())),
        preferred_element_type=jnp.float32)


def _sc_router_body(rlogt_hbm, dmt_hbm, rin, rout):
    wid = lax.axis_index("s") * SC_NC + lax.axis_index("c")
    base = wid * SC_TPW
    pltpu.sync_copy(rlogt_hbm.at[:, pl.ds(base, SC_TPW)], rin)

    kvec = [jnp.full((SC_L,), k, jnp.int32) for k in range(G)]
    for grp in range(SC_TPW // SC_L):
        cols = pl.ds(grp * SC_L, SC_L)
        gl0 = rin[0, cols]
        gl1 = rin[1, cols]
        ll = [rin[2 + k, cols] for k in range(G)]
        gm = jnp.maximum(gl0, gl1)
        ex0 = jnp.exp(gl0 - gm)
        ex1 = jnp.exp(gl1 - gm)
        cw = jnp.maximum(ex0, ex1) / (ex0 + ex1)
        is1 = gl1 > gl0
        lm = jnp.maximum(jnp.maximum(ll[0], ll[1]), jnp.maximum(ll[2], ll[3]))
        le = [jnp.exp(l - lm) for l in ll]
        lsum = (le[0] + le[1]) + (le[2] + le[3])
        lp = [l / lsum for l in le]
        p0 = jnp.maximum(jnp.maximum(lp[0], lp[1]), jnp.maximum(lp[2], lp[3]))
        ti0 = jnp.where(lp[0] == p0, kvec[0],
                        jnp.where(lp[1] == p0, kvec[1],
                                  jnp.where(lp[2] == p0, kvec[2], kvec[3])))
        lpx = [jnp.where(ti0 == kvec[k], -1.0, lp[k]) for k in range(G)]
        p1 = jnp.maximum(jnp.maximum(lpx[0], lpx[1]),
                         jnp.maximum(lpx[2], lpx[3]))
        ti1 = jnp.where(lpx[0] == p1, kvec[0],
                        jnp.where(lpx[1] == p1, kvec[1],
                                  jnp.where(lpx[2] == p1, kvec[2], kvec[3])))
        inv = 1.0 / (p0 + p1 + 1e-7)
        fw0 = cw * p0 * inv
        fw1 = cw * p1 * inv
        for e in range(E):
            l_id = e % G
            wl = (jnp.where(ti0 == kvec[l_id], fw0, 0.0)
                  + jnp.where(ti1 == kvec[l_id], fw1, 0.0))
            if e // G == 1:
                dme = jnp.where(is1, wl, 0.0)
            else:
                dme = jnp.where(is1, 0.0, wl)
            rout[e, cols] = dme
    pltpu.sync_copy(rout, dmt_hbm.at[:, pl.ds(base, SC_TPW)])


def _fused_body(x_ref, dmt_ref, wup_ref, wgate_ref, wdown_ref, wpre_ref,
                wpost_ref, gn_ref, bn_ref,
                adw_ref, adgf_ref, adb_ref, ub_ref, bb_ref,
                wawd_ref, wepwop_ref,
                out_ref,
                sh_s, pre_s, ai_s, ao_s):
    p = pl.program_id(1)
    i = pl.program_id(2)
    rows = pl.ds(i * TF, TF)

    @pl.when(p == 0)
    def _main():
        xt = x_ref[0]
        xb = xt.astype(jnp.bfloat16)
        up = jnp.dot(xb, wup_ref[...], preferred_element_type=jnp.float32)
        gate = jnp.dot(xb, wgate_ref[...], preferred_element_type=jnp.float32)
        hidden = (_silu(gate) * up).astype(jnp.bfloat16)
        sh_s[rows, :] = jnp.dot(hidden, wdown_ref[...],
                                preferred_element_type=jnp.float32)
        pre = jnp.dot(xb, wpre_ref[...], preferred_element_type=jnp.float32)
        pre_s[rows, :] = pre
        g = gn_ref[...]
        b = bn_ref[...]
        ai_s[rows, :] = _ln(pre, g, b)
        po = jnp.dot(hidden, wpost_ref[...],
                     preferred_element_type=jnp.float32)
        ao_s[rows, :] = _ln(po, g, b)

    @pl.when(p == 1)
    def _tail():
        ai_b = ai_s[...]
        ao_b = ao_s[...]
        ai_t = ai_s[rows, :]

        aw = jax.lax.dot_general(ai_t.astype(jnp.bfloat16),
                                 ao_b.astype(jnp.bfloat16),
                                 (((1,), (1,)), ((), ())),
                                 preferred_element_type=jnp.float32)
        aw = _silu(jnp.clip(aw, -5.0, 5.0)).astype(jnp.bfloat16)
        ac = jnp.dot(aw, ai_b.astype(jnp.bfloat16),
                     preferred_element_type=jnp.float32)
        adapt_contrib = jnp.dot(ac, wawd_ref[...],
                                preferred_element_type=jnp.float32)
        shared = sh_s[rows, :] + 0.1 * adapt_contrib

        dm = jnp.transpose(dmt_ref[...])
        wsum = jnp.sum(dm, axis=1, keepdims=True)

        pre_t = pre_s[rows, :]
        P = jnp.dot(pre_t, adw_ref[...],
                    preferred_element_type=jnp.float32)
        sums = jnp.dot(P, ub_ref[...], preferred_element_type=jnp.float32)
        sums2 = jnp.dot(P * P, ub_ref[...],
                        preferred_element_type=jnp.float32)
        mu = sums * (1.0 / A)
        var = sums2 * (1.0 / A) - mu * mu
        rs = jax.lax.rsqrt(var + 1e-5)
        sc = dm * rs
        uc = sc * mu
        Sb = jnp.dot(sc, bb_ref[...], preferred_element_type=jnp.float32)
        Ub = jnp.dot(uc, bb_ref[...], preferred_element_type=jnp.float32)
        zfull = (P * Sb - Ub) * adgf_ref[...]
        z = zfull[:, 0:A]
        for e in range(1, E):
            z = z + zfull[:, e * A:(e + 1) * A]
        z = z + jnp.dot(dm, adb_ref[...], preferred_element_type=jnp.float32)
        expert = jnp.dot(z, wepwop_ref[...],
                         preferred_element_type=jnp.float32)
        out_ref[0] = shared * wsum + 0.1 * expert


@functools.partial(jax.jit, static_argnames=("interpret",))
def _run(x, Wup, Wgate, Wdown, Wpre, Wpost, g_norm, b_norm, Wadapt, adW,
         adg, adb, Wep, Wop, Wrg, Wre, interpret=False):
    xf = x.reshape(N, D)
    wr = jnp.concatenate(
        [Wrg, Wre, jnp.zeros((D, 8 - NG - G), jnp.float32)], axis=1)
    adw_all = jnp.transpose(adW, (1, 0, 2)).reshape(A, E * A)
    gn = g_norm.reshape(1, A)
    bn = b_norm.reshape(1, A)
    ub = jnp.repeat(jnp.eye(E, dtype=jnp.float32), A, axis=0)
    bb = ub.T
    adgf = adg.reshape(1, E * A)
    wup_b = Wup.astype(jnp.bfloat16)
    wgate_b = Wgate.astype(jnp.bfloat16)
    wdown_b = Wdown.astype(jnp.bfloat16)
    wpost_b = Wpost.astype(jnp.bfloat16)
    wpre_b = Wpre.astype(jnp.bfloat16)

    wawd, wepwop = pl.pallas_call(
        _pre_body,
        out_shape=(jax.ShapeDtypeStruct((A, D), jnp.float32),
                   jax.ShapeDtypeStruct((A, D), jnp.float32)),
        interpret=interpret,
    )(Wadapt, Wdown, Wep, Wop)

    nt = N // TF
    rlogt = pl.pallas_call(
        _rlog_body,
        grid=(nt,),
        in_specs=[
            pl.BlockSpec((TF, D), lambda i: (i, 0)),
            pl.BlockSpec((D, 8), lambda i: (0, 0)),
        ],
        out_specs=pl.BlockSpec((8, TF), lambda i: (0, i)),
        out_shape=jax.ShapeDtypeStruct((8, N), jnp.float32),
        interpret=interpret,
    )(xf, wr)

    sc_router = functools.partial(
        pl.kernel,
        mesh=plsc.VectorSubcoreMesh(core_axis_name="c", subcore_axis_name="s"),
        out_type=jax.ShapeDtypeStruct((E, N), jnp.float32),
        scratch_types=[
            pltpu.VMEM((8, SC_TPW), jnp.float32),
            pltpu.VMEM((E, SC_TPW), jnp.float32),
        ],
    )(_sc_router_body)
    dmt = sc_router(rlogt)

    npb = S // TF
    out = pl.pallas_call(
        _fused_body,
        grid=(B, 2, npb),
        in_specs=[
            pl.BlockSpec((1, TF, D), lambda b, p, i: (b, i * (1 - p), 0)),
            pl.BlockSpec((E, TF), lambda b, p, i: (0, (b * (S // TF) + i) * p)),
            pl.BlockSpec((D, H), lambda b, p, i: (0, 0)),
            pl.BlockSpec((D, H), lambda b, p, i: (0, 0)),
            pl.BlockSpec((H, D), lambda b, p, i: (0, 0)),
            pl.BlockSpec((D, A), lambda b, p, i: (0, 0)),
            pl.BlockSpec((H, A), lambda b, p, i: (0, 0)),
            pl.BlockSpec((1, A), lambda b, p, i: (0, 0)),
            pl.BlockSpec((1, A), lambda b, p, i: (0, 0)),
            pl.BlockSpec((A, E * A), lambda b, p, i: (0, 0)),
            pl.BlockSpec((1, E * A), lambda b, p, i: (0, 0)),
            pl.BlockSpec((E, A), lambda b, p, i: (0, 0)),
            pl.BlockSpec((E * A, E), lambda b, p, i: (0, 0)),
            pl.BlockSpec((E, E * A), lambda b, p, i: (0, 0)),
            pl.BlockSpec((A, D), lambda b, p, i: (0, 0)),
            pl.BlockSpec((A, D), lambda b, p, i: (0, 0)),
        ],
        out_specs=pl.BlockSpec((1, TF, D), lambda b, p, i: (b, i * p, 0)),
        out_shape=jax.ShapeDtypeStruct((B, S, D), jnp.float32),
        scratch_shapes=[
            pltpu.VMEM((S, D), jnp.float32),
            pltpu.VMEM((S, A), jnp.float32),
            pltpu.VMEM((S, A), jnp.float32),
            pltpu.VMEM((S, A), jnp.float32),
        ],
        interpret=interpret,
    )(x, dmt, wup_b, wgate_b, wdown_b, wpre_b, wpost_b, gn, bn,
      adw_all, adgf, adb, ub, bb, wawd, wepwop)
    return out


def kernel(x, Wup, Wgate, Wdown, Wpre, Wpost, g_norm, b_norm, Wadapt, adW,
           adg, adb, Wep, Wop, Wrg, Wre):
    return _run(x, Wup, Wgate, Wdown, Wpre, Wpost, g_norm, b_norm, Wadapt,
                adW, adg, adb, Wep, Wop, Wrg, Wre)
